# Initial kernel scaffold; baseline (speedup 1.0000x reference)
import functools

import jax
import jax.numpy as jnp
from jax import lax
from jax.experimental import pallas as pl
from jax.experimental.pallas import tpu as pltpu
from jax.experimental.pallas import tpu_sc as plsc

N = 10000
E = 320000
NB = 1000
GRID = N // NB
NEG = -1e30


def _proj(xw, As_ref, Ad_ref):
    a_s = jnp.dot(xw, As_ref[...], preferred_element_type=jnp.float32)
    a_d = jnp.dot(xw, Ad_ref[...], preferred_element_type=jnp.float32)
    return jnp.concatenate([a_s, a_d], axis=1), jnp.max(a_s)


def _update_c(i, bm, c_ref, cmax_s):
    @pl.when(i == 0)
    def _():
        cmax_s[0] = NEG

    cmax_s[0] = jnp.maximum(cmax_s[0], bm)
    c_ref[...] = jnp.full((1, 16), cmax_s[0], jnp.float32)


def _tc_first_body(x_ref, W_ref, As_ref, Ad_ref, xw_ref, tab_ref, c_ref, cmax_s):
    i = pl.program_id(0)
    xw = jnp.dot(x_ref[...], W_ref[...], preferred_element_type=jnp.float32)
    xw_ref[...] = xw
    tab, bm = _proj(xw, As_ref, Ad_ref)
    tab_ref[...] = tab
    _update_c(i, bm, c_ref, cmax_s)


def _tc_mid_body(acc_ref, b_ref, W_ref, As_ref, Ad_ref, xw_ref, tab_ref, c_ref,
                 cmax_s, *, D, H):
    i = pl.program_id(0)
    num = acc_ref[0, :, :D] + acc_ref[1, :, :D]
    den = acc_ref[0, :, D:D + H] + acc_ref[1, :, D:D + H]
    O = D // H
    denw = jnp.concatenate(
        [jnp.broadcast_to(den[:, h:h + 1], (NB, O)) for h in range(H)], axis=1)
    avg = jnp.where(denw > 0, num / denw, 0.0)
    v = avg + b_ref[...]
    h_act = jnp.where(v > 0, v, jnp.exp(v) - 1.0)
    xw = jnp.dot(h_act, W_ref[...], preferred_element_type=jnp.float32)
    xw_ref[...] = xw
    tab, bm = _proj(xw, As_ref, Ad_ref)
    tab_ref[...] = tab
    _update_c(i, bm, c_ref, cmax_s)


def _tc_final_body(acc_ref, b_ref, out_ref, *, D):
    num = acc_ref[0, :, :D] + acc_ref[1, :, :D]
    den = acc_ref[0, :, D:D + 1] + acc_ref[1, :, D:D + 1]
    denw = jnp.broadcast_to(den, (NB, D))
    logits = jnp.where(denw > 0, num / denw, 0.0) + b_ref[...]
    mx = jnp.max(logits, axis=1, keepdims=True)
    ex = jnp.exp(logits - mx)
    out_ref[...] = ex / jnp.sum(ex, axis=1, keepdims=True)


def _tc_first(x, W, Asm, Adm, D, H):
    return pl.pallas_call(
        _tc_first_body,
        grid=(GRID,),
        in_specs=[
            pl.BlockSpec((NB, x.shape[1]), lambda i: (i, 0)),
            pl.BlockSpec(W.shape, lambda i: (0, 0)),
            pl.BlockSpec(Asm.shape, lambda i: (0, 0)),
            pl.BlockSpec(Adm.shape, lambda i: (0, 0)),
        ],
        out_specs=[
            pl.BlockSpec((NB, D), lambda i: (i, 0)),
            pl.BlockSpec((NB, 2 * H), lambda i: (i, 0)),
            pl.BlockSpec((1, 16), lambda i: (0, 0)),
        ],
        out_shape=[
            jax.ShapeDtypeStruct((N, D), jnp.float32),
            jax.ShapeDtypeStruct((N, 2 * H), jnp.float32),
            jax.ShapeDtypeStruct((1, 16), jnp.float32),
        ],
        scratch_shapes=[pltpu.SMEM((1,), jnp.float32)],
    )(x, W, Asm, Adm)


def _tc_mid(acc, b, W, Asm, Adm, Dprev, Hprev, D, H, WACCp):
    return pl.pallas_call(
        functools.partial(_tc_mid_body, D=Dprev, H=Hprev),
        grid=(GRID,),
        in_specs=[
            pl.BlockSpec((2, NB, WACCp), lambda i: (0, i, 0)),
            pl.BlockSpec((1, Dprev), lambda i: (0, 0)),
            pl.BlockSpec(W.shape, lambda i: (0, 0)),
            pl.BlockSpec(Asm.shape, lambda i: (0, 0)),
            pl.BlockSpec(Adm.shape, lambda i: (0, 0)),
        ],
        out_specs=[
            pl.BlockSpec((NB, D), lambda i: (i, 0)),
            pl.BlockSpec((NB, 2 * H), lambda i: (i, 0)),
            pl.BlockSpec((1, 16), lambda i: (0, 0)),
        ],
        out_shape=[
            jax.ShapeDtypeStruct((N, D), jnp.float32),
            jax.ShapeDtypeStruct((N, 2 * H), jnp.float32),
            jax.ShapeDtypeStruct((1, 16), jnp.float32),
        ],
        scratch_shapes=[pltpu.SMEM((1,), jnp.float32)],
    )(acc, b, W, Asm, Adm)


def _tc_final(acc, b, D, WACCp):
    return pl.pallas_call(
        functools.partial(_tc_final_body, D=D),
        grid=(GRID,),
        in_specs=[
            pl.BlockSpec((2, NB, WACCp), lambda i: (0, i, 0)),
            pl.BlockSpec((1, D), lambda i: (0, 0)),
        ],
        out_specs=pl.BlockSpec((NB, D), lambda i: (i, 0)),
        out_shape=jax.ShapeDtypeStruct((N, D), jnp.float32),
    )(acc, b)


BLK = 80
EPW = E // 32
RPT = N // 16


def _sc_edge_kernel(H, D, WACC):
    O = D // H
    mesh = plsc.VectorSubcoreMesh(core_axis_name="c", subcore_axis_name="s")

    @functools.partial(
        pl.kernel,
        out_type=jax.ShapeDtypeStruct((2, N, WACC), jnp.float32),
        mesh=mesh,
        scratch_types=[
            pltpu.VMEM((N, 2 * H), jnp.float32),
            pltpu.VMEM((BLK,), jnp.int32),
            pltpu.VMEM((BLK,), jnp.int32),
            pltpu.VMEM((BLK, D), jnp.float32),
            pltpu.VMEM((BLK, WACC), jnp.float32),
            pltpu.VMEM((16,), jnp.float32),
            pltpu.VMEM_SHARED((N, WACC), jnp.float32),
        ],
    )
    def k(xw_hbm, tab_hbm, c_hbm, src_hbm, dst_hbm, zeros_hbm, out_hbm,
          tab_v, esrc_v, edst_v, rows_v, msg_v, c_v, acc_sh):
        cid = lax.axis_index("c")
        sid = lax.axis_index("s")
        wid = cid * 16 + sid

        pltpu.sync_copy(zeros_hbm.at[pl.ds(sid * RPT, RPT)],
                        acc_sh.at[pl.ds(sid * RPT, RPT)])
        pltpu.sync_copy(tab_hbm, tab_v)
        pltpu.sync_copy(c_hbm, c_v)

        ids = lax.iota(jnp.int32, 16)
        zero16 = jnp.zeros((16,), jnp.float32)
        for col in range(D + H, WACC):
            colv = jnp.full((16,), col, jnp.int32)

            @pl.loop(0, BLK, step=16)
            def _(kk):
                plsc.store_scatter(msg_v, [kk + ids, colv], zero16)

        plsc.subcore_barrier()

        cvec = jnp.full((16,), c_v[0], jnp.float32)
        base0 = wid * EPW

        @pl.loop(0, EPW, step=BLK)
        def _(off):
            base = base0 + off
            pltpu.sync_copy(src_hbm.at[pl.ds(base, BLK)], esrc_v)
            pltpu.sync_copy(dst_hbm.at[pl.ds(base, BLK)], edst_v)
            pltpu.sync_copy(xw_hbm.at[esrc_v], rows_v)

            @pl.loop(0, BLK, step=16)
            def _(i):
                src = esrc_v[pl.ds(i, 16)]
                dst = edst_v[pl.ds(i, 16)]
                eids = i + ids
                for h in range(H):
                    hs = plsc.load_gather(
                        tab_v, [src, jnp.full((16,), h, jnp.int32)])
                    hd = plsc.load_gather(
                        tab_v, [dst, jnp.full((16,), H + h, jnp.int32)])
                    s = hs + hd
                    e = jnp.maximum(s, 0.2 * s)
                    t = cvec + hd
                    mp = jnp.maximum(t, 0.2 * t)
                    ex = jnp.exp(e - mp)
                    for col in range(h * O, (h + 1) * O):
                        colv = jnp.full((16,), col, jnp.int32)
                        v = plsc.load_gather(rows_v, [eids, colv])
                        plsc.store_scatter(msg_v, [eids, colv], v * ex)
                    plsc.store_scatter(
                        msg_v, [eids, jnp.full((16,), D + h, jnp.int32)], ex)

            pltpu.sync_copy(msg_v, acc_sh.at[edst_v], add=True)

        plsc.subcore_barrier()
        pltpu.sync_copy(acc_sh.at[pl.ds(sid * RPT, RPT)],
                        out_hbm.at[cid, pl.ds(sid * RPT, RPT)])

    return k


def _blockdiag(a):
    Hh, Oo = a.shape
    m = jnp.zeros((Hh * Oo, Hh), a.dtype)
    for h in range(Hh):
        m = m.at[h * Oo:(h + 1) * Oo, h].set(a[h])
    return m


_sc_l01 = _sc_edge_kernel(H=2, D=64, WACC=72)
_sc_l2 = _sc_edge_kernel(H=1, D=40, WACC=48)


def kernel(x, edge_index, W0, a_src0, a_dst0, b0, W1, a_src1, a_dst1, b1,
           W2, a_src2, a_dst2, b2):
    src = edge_index[0]
    dst = edge_index[1]
    z72 = jnp.zeros((N, 72), jnp.float32)
    z48 = jnp.zeros((N, 48), jnp.float32)

    xw, tab, c = _tc_first(x, W0, _blockdiag(a_src0), _blockdiag(a_dst0),
                           D=64, H=2)
    acc = _sc_l01(xw, tab, c.reshape(16), src, dst, z72)

    xw, tab, c = _tc_mid(acc, b0.reshape(1, 64), W1, _blockdiag(a_src1),
                         _blockdiag(a_dst1), Dprev=64, Hprev=2, D=64, H=2,
                         WACCp=72)
    acc = _sc_l01(xw, tab, c.reshape(16), src, dst, z72)

    xw, tab, c = _tc_mid(acc, b1.reshape(1, 64), W2, _blockdiag(a_src2),
                         _blockdiag(a_dst2), Dprev=64, Hprev=2, D=40, H=1,
                         WACCp=72)
    acc = _sc_l2(xw, tab, c.reshape(16), src, dst, z48)

    return _tc_final(acc, b2.reshape(1, 40), D=40, WACCp=48)


# speedup vs baseline: 15.1865x; 15.1865x over previous
import functools

import jax
import jax.numpy as jnp
from jax import lax
from jax.experimental import pallas as pl
from jax.experimental.pallas import tpu as pltpu
from jax.experimental.pallas import tpu_sc as plsc

N = 10000
NP = 10240
E = 320000
NB = 1024
GRID = NP // NB
NEG = -1e30


def _proj(xw, As_ref, Ad_ref):
    a_s = jnp.dot(xw, As_ref[...], preferred_element_type=jnp.float32)
    a_d = jnp.dot(xw, Ad_ref[...], preferred_element_type=jnp.float32)
    return jnp.concatenate([a_s, a_d], axis=1), jnp.max(a_s)


def _update_c(i, bm, c_ref, cmax_s):
    @pl.when(i == 0)
    def _():
        cmax_s[0] = NEG

    cmax_s[0] = jnp.maximum(cmax_s[0], bm)
    c_ref[...] = jnp.full((1, 16), cmax_s[0], jnp.float32)


def _tc_first_body(x_ref, W_ref, As_ref, Ad_ref, xw_ref, tab_ref, c_ref, cmax_s):
    i = pl.program_id(0)
    xw = jnp.dot(x_ref[...], W_ref[...], preferred_element_type=jnp.float32)
    xw_ref[...] = xw
    tab, bm = _proj(xw, As_ref, Ad_ref)
    tab_ref[...] = tab
    _update_c(i, bm, c_ref, cmax_s)


def _tc_mid_body(acc_ref, b_ref, W_ref, As_ref, Ad_ref, xw_ref, tab_ref, c_ref,
                 cmax_s, *, D, H):
    i = pl.program_id(0)
    num = acc_ref[0, :, :D] + acc_ref[1, :, :D]
    den = acc_ref[0, :, D:D + H] + acc_ref[1, :, D:D + H]
    O = D // H
    denw = jnp.concatenate(
        [jnp.broadcast_to(den[:, h:h + 1], (NB, O)) for h in range(H)], axis=1)
    avg = jnp.where(denw > 0, num / denw, 0.0)
    v = avg + b_ref[...]
    h_act = jnp.where(v > 0, v, jnp.exp(v) - 1.0)
    xw = jnp.dot(h_act, W_ref[...], preferred_element_type=jnp.float32)
    xw_ref[...] = xw
    tab, bm = _proj(xw, As_ref, Ad_ref)
    tab_ref[...] = tab
    _update_c(i, bm, c_ref, cmax_s)


def _tc_final_body(acc_ref, b_ref, out_ref, *, D):
    num = acc_ref[0, :, :D] + acc_ref[1, :, :D]
    den = acc_ref[0, :, D:D + 1] + acc_ref[1, :, D:D + 1]
    denw = jnp.broadcast_to(den, (NB, D))
    logits = jnp.where(denw > 0, num / denw, 0.0) + b_ref[...]
    mx = jnp.max(logits, axis=1, keepdims=True)
    ex = jnp.exp(logits - mx)
    out_ref[...] = ex / jnp.sum(ex, axis=1, keepdims=True)


def _tc_first(x, W, Asm, Adm, D, H):
    return pl.pallas_call(
        _tc_first_body,
        grid=(GRID,),
        in_specs=[
            pl.BlockSpec((NB, x.shape[1]), lambda i: (i, 0)),
            pl.BlockSpec(W.shape, lambda i: (0, 0)),
            pl.BlockSpec(Asm.shape, lambda i: (0, 0)),
            pl.BlockSpec(Adm.shape, lambda i: (0, 0)),
        ],
        out_specs=[
            pl.BlockSpec((NB, D), lambda i: (i, 0)),
            pl.BlockSpec((NB, 2 * H), lambda i: (i, 0)),
            pl.BlockSpec((1, 16), lambda i: (0, 0)),
        ],
        out_shape=[
            jax.ShapeDtypeStruct((NP, D), jnp.float32),
            jax.ShapeDtypeStruct((NP, 2 * H), jnp.float32),
            jax.ShapeDtypeStruct((1, 16), jnp.float32),
        ],
        scratch_shapes=[pltpu.SMEM((1,), jnp.float32)],
    )(x, W, Asm, Adm)


def _tc_mid(acc, b, W, Asm, Adm, Dprev, Hprev, D, H):
    return pl.pallas_call(
        functools.partial(_tc_mid_body, D=Dprev, H=Hprev),
        grid=(GRID,),
        in_specs=[
            pl.BlockSpec((2, NB, 128), lambda i: (0, i, 0)),
            pl.BlockSpec((1, Dprev), lambda i: (0, 0)),
            pl.BlockSpec(W.shape, lambda i: (0, 0)),
            pl.BlockSpec(Asm.shape, lambda i: (0, 0)),
            pl.BlockSpec(Adm.shape, lambda i: (0, 0)),
        ],
        out_specs=[
            pl.BlockSpec((NB, D), lambda i: (i, 0)),
            pl.BlockSpec((NB, 2 * H), lambda i: (i, 0)),
            pl.BlockSpec((1, 16), lambda i: (0, 0)),
        ],
        out_shape=[
            jax.ShapeDtypeStruct((NP, D), jnp.float32),
            jax.ShapeDtypeStruct((NP, 2 * H), jnp.float32),
            jax.ShapeDtypeStruct((1, 16), jnp.float32),
        ],
        scratch_shapes=[pltpu.SMEM((1,), jnp.float32)],
    )(acc, b, W, Asm, Adm)


def _tc_final(acc, b, D):
    return pl.pallas_call(
        functools.partial(_tc_final_body, D=D),
        grid=(GRID,),
        in_specs=[
            pl.BlockSpec((2, NB, 128), lambda i: (0, i, 0)),
            pl.BlockSpec((1, D), lambda i: (0, 0)),
        ],
        out_specs=pl.BlockSpec((NB, D), lambda i: (i, 0)),
        out_shape=jax.ShapeDtypeStruct((NP, D), jnp.float32),
    )(acc, b)


BLK = 80
EPW = E // 32
RPT = NP // 16
WACC = 128


def _sc_edge_kernel(H, D):
    O = D // H
    mesh = plsc.VectorSubcoreMesh(core_axis_name="c", subcore_axis_name="s")
    cparams = pltpu.CompilerParams(needs_layout_passes=False)

    @functools.partial(
        pl.kernel,
        out_type=jax.ShapeDtypeStruct((2, NP, WACC), jnp.float32),
        mesh=mesh,
        compiler_params=cparams,
        scratch_types=[
            pltpu.VMEM((BLK,), jnp.int32),
            pltpu.VMEM((BLK,), jnp.int32),
            [pltpu.VMEM((BLK,), jnp.int32) for _ in range(2 * H)],
            [pltpu.VMEM((BLK,), jnp.float32) for _ in range(2 * H)],
            pltpu.VMEM((BLK, WACC), jnp.float32),
            pltpu.VMEM((BLK, WACC), jnp.float32),
            pltpu.VMEM((16,), jnp.float32),
            pltpu.VMEM_SHARED((NP, WACC), jnp.float32),
        ],
    )
    def k(xw_hbm, tab_hbm, c_hbm, src_hbm, dst_hbm, zeros_hbm, out_hbm,
          esrc_v, edst_v, tidx_v, tval_v, rows_v, msg_v, c_v, acc_sh):
        cid = lax.axis_index("c")
        sid = lax.axis_index("s")
        wid = cid * 16 + sid

        pltpu.sync_copy(zeros_hbm.at[pl.ds(sid * RPT, RPT)],
                        acc_sh.at[pl.ds(sid * RPT, RPT)])
        pltpu.sync_copy(c_hbm, c_v)

        ids = lax.iota(jnp.int32, 16)
        zero16 = jnp.zeros((16,), jnp.float32)
        for col in range(D + H, WACC):
            colv = jnp.full((16,), col, jnp.int32)

            @pl.loop(0, BLK, step=16)
            def _(kk):
                plsc.store_scatter(msg_v, [kk + ids, colv], zero16)

        plsc.subcore_barrier()

        cvec = c_v[...]
        base0 = wid * EPW

        @pl.loop(0, EPW, step=BLK)
        def _(off):
            base = base0 + off
            pltpu.sync_copy(src_hbm.at[pl.ds(base, BLK)], esrc_v)
            pltpu.sync_copy(dst_hbm.at[pl.ds(base, BLK)], edst_v)

            @pl.loop(0, BLK, step=16)
            def _(i):
                s16 = esrc_v[pl.ds(i, 16)] * (2 * H)
                d16 = edst_v[pl.ds(i, 16)] * (2 * H)
                for h in range(H):
                    plsc.store_scatter(tidx_v[h], [i + ids], s16 + h)
                    plsc.store_scatter(tidx_v[H + h], [i + ids], d16 + (H + h))

            for t in range(2 * H):
                pltpu.sync_copy(tab_hbm.at[tidx_v[t]], tval_v[t])
            pltpu.sync_copy(xw_hbm.at[esrc_v], rows_v)

            @pl.loop(0, BLK, step=16)
            def _(i):
                eids = i + ids
                for h in range(H):
                    hs = tval_v[h][pl.ds(i, 16)]
                    hd = tval_v[H + h][pl.ds(i, 16)]
                    s = hs + hd
                    e = jnp.maximum(s, 0.2 * s)
                    t = cvec + hd
                    mp = jnp.maximum(t, 0.2 * t)
                    ex = jnp.exp(e - mp)
                    for col in range(h * O, (h + 1) * O):
                        colv = jnp.full((16,), col, jnp.int32)
                        v = plsc.load_gather(rows_v, [eids, colv])
                        plsc.store_scatter(msg_v, [eids, colv], v * ex)
                    plsc.store_scatter(
                        msg_v, [eids, jnp.full((16,), D + h, jnp.int32)], ex)

            pltpu.sync_copy(msg_v, acc_sh.at[edst_v], add=True)

        plsc.subcore_barrier()
        pltpu.sync_copy(acc_sh.at[pl.ds(sid * RPT, RPT)],
                        out_hbm.at[cid, pl.ds(sid * RPT, RPT)])

    return k


def _blockdiag(a):
    Hh, Oo = a.shape
    m = jnp.zeros((128, Hh), a.dtype)
    for h in range(Hh):
        m = m.at[h * Oo:(h + 1) * Oo, h].set(a[h])
    return m


_sc_l01 = _sc_edge_kernel(H=2, D=64)
_sc_l2 = _sc_edge_kernel(H=1, D=40)


def kernel(x, edge_index, W0, a_src0, a_dst0, b0, W1, a_src1, a_dst1, b1,
           W2, a_src2, a_dst2, b2):
    src = edge_index[0]
    dst = edge_index[1]
    x = jnp.pad(x, ((0, NP - N), (0, 0)))
    zeros = jnp.zeros((NP, 128), jnp.float32)
    W0p = jnp.pad(W0, ((0, 0), (0, 64)))
    W1p = jnp.pad(W1, ((0, 0), (0, 64)))
    W2p = jnp.pad(W2, ((0, 0), (0, 88)))

    xw, tab, c = _tc_first(x, W0p, _blockdiag(a_src0), _blockdiag(a_dst0),
                           D=128, H=2)
    acc = _sc_l01(xw, tab.reshape(NP * 4), c.reshape(16), src, dst, zeros)

    xw, tab, c = _tc_mid(acc, b0.reshape(1, 64), W1p, _blockdiag(a_src1),
                         _blockdiag(a_dst1), Dprev=64, Hprev=2, D=128, H=2)
    acc = _sc_l01(xw, tab.reshape(NP * 4), c.reshape(16), src, dst, zeros)

    xw, tab, c = _tc_mid(acc, b1.reshape(1, 64), W2p, _blockdiag(a_src2),
                         _blockdiag(a_dst2), Dprev=64, Hprev=2, D=128, H=1)
    acc = _sc_l2(xw, tab.reshape(NP * 2), c.reshape(16), src, dst, zeros)

    return _tc_final(acc, b2.reshape(1, 40), D=40)[:N]
